# Initial kernel scaffold; baseline (speedup 1.0000x reference)
import jax
import jax.numpy as jnp
from jax.experimental import pallas as pl


def kernel(x, edge_index, W1, b1, W2, b2, Wc, bc):
    raise NotImplementedError("write your pallas kernel here")


# speedup vs baseline: 11.4019x; 11.4019x over previous
import functools

import jax
import jax.numpy as jnp
from jax import lax
from jax.experimental import pallas as pl
from jax.experimental.pallas import tpu as pltpu
from jax.experimental.pallas import tpu_sc as plsc

N = 10000
E = 320000
D_IN = 128
D_H = 128
D_OUT = 64

NC = 2
NS = 16
NT = NC * NS
EPT = E // NT
K = 80
NCHUNK = EPT // K
RPT = N // NS
HL = 16
ZR = 125

def _hist_body(srcr_hbm, dstr_hbm, out_hbm, src_v, dst_v, ones_v, zero_v,
               ha_sh, hb_sh):
    c = lax.axis_index("c")
    s = lax.axis_index("s")

    def fill_ones(i, carry):
        ones_v[i, :] = jnp.ones((HL,), jnp.float32)
        return carry

    lax.fori_loop(0, K, fill_ones, 0)

    def fill_zero(i, carry):
        zero_v[i, :] = jnp.zeros((HL,), jnp.float32)
        return carry

    lax.fori_loop(0, RPT, fill_zero, 0)

    pltpu.sync_copy(zero_v, ha_sh.at[pl.ds(s * RPT, RPT)])
    pltpu.sync_copy(zero_v, hb_sh.at[pl.ds(s * RPT, RPT)])

    pltpu.sync_copy(srcr_hbm.at[c, s], src_v)
    pltpu.sync_copy(dstr_hbm.at[c, s], dst_v)
    plsc.subcore_barrier()

    def body(j, carry):
        pltpu.sync_copy(ones_v, ha_sh.at[src_v.at[j]], add=True)
        pltpu.sync_copy(ones_v, hb_sh.at[dst_v.at[j]], add=True)
        return carry

    lax.fori_loop(0, NCHUNK, body, 0)
    plsc.subcore_barrier()

    pltpu.sync_copy(ha_sh.at[pl.ds(s * RPT, RPT)],
                    out_hbm.at[c, 0, pl.ds(s * RPT, RPT)])
    pltpu.sync_copy(hb_sh.at[pl.ds(s * RPT, RPT)],
                    out_hbm.at[c, 1, pl.ds(s * RPT, RPT)])


def _agg_body(hs_hbm, srcr_hbm, dstr_hbm, out_hbm, src_v, dst_v, rows_v,
              zero_v, acc_sh, sem):
    c = lax.axis_index("c")
    s = lax.axis_index("s")

    def fill_zero(i, carry):
        r = i // 8
        l = (i % 8) * 16
        zero_v[r, pl.ds(l, 16)] = jnp.zeros((16,), jnp.float32)
        return carry

    lax.fori_loop(0, ZR * 8, fill_zero, 0)

    for z in range(RPT // ZR):
        pltpu.sync_copy(zero_v, acc_sh.at[pl.ds(s * RPT + z * ZR, ZR)])

    pltpu.sync_copy(srcr_hbm.at[c, s], src_v)
    pltpu.sync_copy(dstr_hbm.at[c, s], dst_v)
    plsc.subcore_barrier()

    def body(j, carry):
        pltpu.async_copy(hs_hbm.at[src_v.at[j]], rows_v, sem).wait()
        pltpu.sync_copy(rows_v, acc_sh.at[dst_v.at[j]], add=True)
        return carry

    lax.fori_loop(0, NCHUNK, body, 0)
    plsc.subcore_barrier()

    pltpu.sync_copy(acc_sh.at[pl.ds(s * RPT, RPT)],
                    out_hbm.at[c, pl.ds(s * RPT, RPT)])


@functools.cache
def _sc_kernels():
    mesh = plsc.VectorSubcoreMesh(core_axis_name="c", subcore_axis_name="s",
                                  num_cores=NC, num_subcores=NS)
    params = pltpu.CompilerParams(use_tc_tiling_on_sc=False)
    hist = pl.kernel(
        _hist_body,
        out_type=jax.ShapeDtypeStruct((NC, 2, N, HL), jnp.float32),
        mesh=mesh,
        compiler_params=params,
        scratch_types=[
            pltpu.VMEM((NCHUNK, K), jnp.int32),
            pltpu.VMEM((NCHUNK, K), jnp.int32),
            pltpu.VMEM((K, HL), jnp.float32),
            pltpu.VMEM((RPT, HL), jnp.float32),
            pltpu.VMEM_SHARED((N, HL), jnp.float32),
            pltpu.VMEM_SHARED((N, HL), jnp.float32),
        ],
    )
    agg = pl.kernel(
        _agg_body,
        out_type=jax.ShapeDtypeStruct((NC, N, D_H), jnp.float32),
        mesh=mesh,
        compiler_params=params,
        scratch_types=[
            pltpu.VMEM((NCHUNK, K), jnp.int32),
            pltpu.VMEM((NCHUNK, K), jnp.int32),
            pltpu.VMEM((K, D_H), jnp.float32),
            pltpu.VMEM((ZR, D_H), jnp.float32),
            pltpu.VMEM_SHARED((N, D_H), jnp.float32),
            pltpu.SemaphoreType.DMA,
        ],
    )
    return hist, agg


_BR = 1000
_GRID = N // _BR


def _norms(degs):
    deg_out = jnp.sum(degs[:, 0, :, 0], axis=0)
    deg_in = jnp.sum(degs[:, 1, :, 0], axis=0)
    ns = lax.rsqrt(jnp.maximum(deg_out, 1.0))
    nd = lax.rsqrt(jnp.maximum(deg_in, 1.0))
    return ns, nd


def _mm1_body(x_ref, degs_ref, w_ref, out_ref):
    ns, _ = _norms(degs_ref[...])
    out_ref[...] = jnp.dot(x_ref[...] * ns[:, None], w_ref[...],
                           preferred_element_type=jnp.float32)


def _mm1(x, degs, w):
    return pl.pallas_call(
        _mm1_body,
        grid=(_GRID,),
        in_specs=[
            pl.BlockSpec((_BR, D_IN), lambda i: (i, 0)),
            pl.BlockSpec((NC, 2, _BR, HL), lambda i: (0, 0, i, 0)),
            pl.BlockSpec((D_IN, D_H), lambda i: (0, 0)),
        ],
        out_specs=pl.BlockSpec((_BR, D_H), lambda i: (i, 0)),
        out_shape=jax.ShapeDtypeStruct((N, D_H), jnp.float32),
    )(x, degs, w)


def _mid_body(agg_ref, degs_ref, b_ref, w_ref, out_ref):
    ns, nd = _norms(degs_ref[...])
    a = agg_ref[0] + agg_ref[1]
    h = jnp.maximum(a * nd[:, None] + b_ref[...], 0.0)
    out_ref[...] = jnp.dot(h * ns[:, None], w_ref[...],
                           preferred_element_type=jnp.float32)


def _mid(agg, degs, b, w):
    return pl.pallas_call(
        _mid_body,
        grid=(_GRID,),
        in_specs=[
            pl.BlockSpec((NC, _BR, D_H), lambda i: (0, i, 0)),
            pl.BlockSpec((NC, 2, _BR, HL), lambda i: (0, 0, i, 0)),
            pl.BlockSpec((1, D_H), lambda i: (0, 0)),
            pl.BlockSpec((D_H, D_H), lambda i: (0, 0)),
        ],
        out_specs=pl.BlockSpec((_BR, D_H), lambda i: (i, 0)),
        out_shape=jax.ShapeDtypeStruct((N, D_H), jnp.float32),
    )(agg, degs, b, w)


def _fin_body(agg_ref, degs_ref, b_ref, wc_ref, bc_ref, out_ref):
    _, nd = _norms(degs_ref[...])
    a = agg_ref[0] + agg_ref[1]
    h = jnp.maximum(a * nd[:, None] + b_ref[...], 0.0)
    out_ref[...] = jnp.dot(h, wc_ref[...],
                           preferred_element_type=jnp.float32) + bc_ref[...]


def _fin(agg, degs, b, wc, bc):
    return pl.pallas_call(
        _fin_body,
        grid=(_GRID,),
        in_specs=[
            pl.BlockSpec((NC, _BR, D_H), lambda i: (0, i, 0)),
            pl.BlockSpec((NC, 2, _BR, HL), lambda i: (0, 0, i, 0)),
            pl.BlockSpec((1, D_H), lambda i: (0, 0)),
            pl.BlockSpec((D_H, D_OUT), lambda i: (0, 0)),
            pl.BlockSpec((1, D_OUT), lambda i: (0, 0)),
        ],
        out_specs=pl.BlockSpec((_BR, D_OUT), lambda i: (i, 0)),
        out_shape=jax.ShapeDtypeStruct((N, D_OUT), jnp.float32),
    )(agg, degs, b, wc, bc)


def kernel(x, edge_index, W1, b1, W2, b2, Wc, bc):
    src = edge_index[0].astype(jnp.int32)
    dst = edge_index[1].astype(jnp.int32)
    srcr = src.reshape(NC, NS, NCHUNK, K)
    dstr = dst.reshape(NC, NS, NCHUNK, K)

    hist_kernel, agg_kernel = _sc_kernels()
    degs = hist_kernel(srcr, dstr)
    hs1 = _mm1(x, degs, W1)
    agg1 = agg_kernel(hs1, srcr, dstr)
    hs2 = _mid(agg1, degs, b1.reshape(1, D_H), W2)
    agg2 = agg_kernel(hs2, srcr, dstr)
    return _fin(agg2, degs, b2.reshape(1, D_H), Wc, bc.reshape(1, D_OUT))


# speedup vs baseline: 16.7861x; 1.4722x over previous
import functools

import jax
import jax.numpy as jnp
from jax import lax
from jax.experimental import pallas as pl
from jax.experimental.pallas import tpu as pltpu
from jax.experimental.pallas import tpu_sc as plsc

N = 10000
E = 320000
D_IN = 128
D_H = 128
D_OUT = 64

NC = 2
NS = 16
NT = NC * NS
EPT = E // NT
K = 40
NCHUNK = EPT // K
RPT = N // NS
HL = 16
ZR = 25

def _hist_body(srcr_hbm, dstr_hbm, out_hbm, src_v, dst_v, ones_v, zero_v,
               ha_sh, hb_sh):
    c = lax.axis_index("c")
    s = lax.axis_index("s")

    def fill_ones(i, carry):
        ones_v[i, :] = jnp.ones((HL,), jnp.float32)
        return carry

    lax.fori_loop(0, K, fill_ones, 0)

    def fill_zero(i, carry):
        zero_v[i, :] = jnp.zeros((HL,), jnp.float32)
        return carry

    lax.fori_loop(0, RPT, fill_zero, 0)

    pltpu.sync_copy(zero_v, ha_sh.at[pl.ds(s * RPT, RPT)])
    pltpu.sync_copy(zero_v, hb_sh.at[pl.ds(s * RPT, RPT)])

    pltpu.sync_copy(srcr_hbm.at[c, s], src_v)
    pltpu.sync_copy(dstr_hbm.at[c, s], dst_v)
    plsc.subcore_barrier()

    def body(j, carry):
        pltpu.sync_copy(ones_v, ha_sh.at[src_v.at[j]], add=True)
        pltpu.sync_copy(ones_v, hb_sh.at[dst_v.at[j]], add=True)
        return carry

    lax.fori_loop(0, NCHUNK, body, 0)
    plsc.subcore_barrier()

    pltpu.sync_copy(ha_sh.at[pl.ds(s * RPT, RPT)],
                    out_hbm.at[c, 0, pl.ds(s * RPT, RPT)])
    pltpu.sync_copy(hb_sh.at[pl.ds(s * RPT, RPT)],
                    out_hbm.at[c, 1, pl.ds(s * RPT, RPT)])


NB = 5
ROUNDS = NCHUNK // NB


def _agg_body(hs_hbm, srcr_hbm, dstr_hbm, out_hbm, src_v, dst_v,
              rows0, rows1, rows2, rows3, rows4, zero_v, acc_sh, *sems):
    rows = (rows0, rows1, rows2, rows3, rows4)
    gsem = sems[:NB]
    ssem = sems[NB:]
    c = lax.axis_index("c")
    s = lax.axis_index("s")

    pltpu.sync_copy(srcr_hbm.at[c, s], src_v)
    pltpu.sync_copy(dstr_hbm.at[c, s], dst_v)
    for b in range(NB):
        pltpu.async_copy(hs_hbm.at[src_v.at[b]], rows[b], gsem[b])

    def fill_zero(i, carry):
        r = i // 8
        l = (i % 8) * 16
        zero_v[r, pl.ds(l, 16)] = jnp.zeros((16,), jnp.float32)
        return carry

    lax.fori_loop(0, ZR * 8, fill_zero, 0)

    for z in range(RPT // ZR):
        pltpu.sync_copy(zero_v, acc_sh.at[pl.ds(s * RPT + z * ZR, ZR)])
    plsc.subcore_barrier()

    def round_body(r, carry):
        for b in range(NB):
            j = r * NB + b
            pltpu.make_async_copy(hs_hbm.at[src_v.at[j]], rows[b],
                                  gsem[b]).wait()
            pltpu.async_copy(rows[b], acc_sh.at[dst_v.at[j]], ssem[b],
                             add=True)
        for b in range(NB):
            j = r * NB + b
            jn = j + NB
            pltpu.make_async_copy(rows[b], acc_sh.at[dst_v.at[j]],
                                  ssem[b]).wait()

            @pl.when(jn < NCHUNK)
            def _():
                pltpu.async_copy(hs_hbm.at[src_v.at[jn]], rows[b], gsem[b])

        return carry

    lax.fori_loop(0, ROUNDS, round_body, 0)
    plsc.subcore_barrier()

    pltpu.sync_copy(acc_sh.at[pl.ds(s * RPT, RPT)],
                    out_hbm.at[c, pl.ds(s * RPT, RPT)])


@functools.cache
def _sc_kernels():
    mesh = plsc.VectorSubcoreMesh(core_axis_name="c", subcore_axis_name="s",
                                  num_cores=NC, num_subcores=NS)
    params = pltpu.CompilerParams(use_tc_tiling_on_sc=False)
    hist = pl.kernel(
        _hist_body,
        out_type=jax.ShapeDtypeStruct((NC, 2, N, HL), jnp.float32),
        mesh=mesh,
        compiler_params=params,
        scratch_types=[
            pltpu.VMEM((NCHUNK, K), jnp.int32),
            pltpu.VMEM((NCHUNK, K), jnp.int32),
            pltpu.VMEM((K, HL), jnp.float32),
            pltpu.VMEM((RPT, HL), jnp.float32),
            pltpu.VMEM_SHARED((N, HL), jnp.float32),
            pltpu.VMEM_SHARED((N, HL), jnp.float32),
        ],
    )
    agg = pl.kernel(
        _agg_body,
        out_type=jax.ShapeDtypeStruct((NC, N, D_H), jnp.float32),
        mesh=mesh,
        compiler_params=params,
        scratch_types=[
            pltpu.VMEM((NCHUNK, K), jnp.int32),
            pltpu.VMEM((NCHUNK, K), jnp.int32),
        ] + [pltpu.VMEM((K, D_H), jnp.float32) for _ in range(NB)] + [
            pltpu.VMEM((ZR, D_H), jnp.float32),
            pltpu.VMEM_SHARED((N, D_H), jnp.float32),
        ] + [pltpu.SemaphoreType.DMA for _ in range(2 * NB)],
    )
    return hist, agg


_BR = 1000
_GRID = N // _BR


def _norms(degs):
    deg_out = jnp.sum(degs[:, 0, :, 0], axis=0)
    deg_in = jnp.sum(degs[:, 1, :, 0], axis=0)
    ns = lax.rsqrt(jnp.maximum(deg_out, 1.0))
    nd = lax.rsqrt(jnp.maximum(deg_in, 1.0))
    return ns, nd


def _mm1_body(x_ref, degs_ref, w_ref, out_ref):
    ns, _ = _norms(degs_ref[...])
    out_ref[...] = jnp.dot(x_ref[...] * ns[:, None], w_ref[...],
                           preferred_element_type=jnp.float32)


def _mm1(x, degs, w):
    return pl.pallas_call(
        _mm1_body,
        grid=(_GRID,),
        in_specs=[
            pl.BlockSpec((_BR, D_IN), lambda i: (i, 0)),
            pl.BlockSpec((NC, 2, _BR, HL), lambda i: (0, 0, i, 0)),
            pl.BlockSpec((D_IN, D_H), lambda i: (0, 0)),
        ],
        out_specs=pl.BlockSpec((_BR, D_H), lambda i: (i, 0)),
        out_shape=jax.ShapeDtypeStruct((N, D_H), jnp.float32),
    )(x, degs, w)


def _mid_body(agg_ref, degs_ref, b_ref, w_ref, out_ref):
    ns, nd = _norms(degs_ref[...])
    a = agg_ref[0] + agg_ref[1]
    h = jnp.maximum(a * nd[:, None] + b_ref[...], 0.0)
    out_ref[...] = jnp.dot(h * ns[:, None], w_ref[...],
                           preferred_element_type=jnp.float32)


def _mid(agg, degs, b, w):
    return pl.pallas_call(
        _mid_body,
        grid=(_GRID,),
        in_specs=[
            pl.BlockSpec((NC, _BR, D_H), lambda i: (0, i, 0)),
            pl.BlockSpec((NC, 2, _BR, HL), lambda i: (0, 0, i, 0)),
            pl.BlockSpec((1, D_H), lambda i: (0, 0)),
            pl.BlockSpec((D_H, D_H), lambda i: (0, 0)),
        ],
        out_specs=pl.BlockSpec((_BR, D_H), lambda i: (i, 0)),
        out_shape=jax.ShapeDtypeStruct((N, D_H), jnp.float32),
    )(agg, degs, b, w)


def _fin_body(agg_ref, degs_ref, b_ref, wc_ref, bc_ref, out_ref):
    _, nd = _norms(degs_ref[...])
    a = agg_ref[0] + agg_ref[1]
    h = jnp.maximum(a * nd[:, None] + b_ref[...], 0.0)
    out_ref[...] = jnp.dot(h, wc_ref[...],
                           preferred_element_type=jnp.float32) + bc_ref[...]


def _fin(agg, degs, b, wc, bc):
    return pl.pallas_call(
        _fin_body,
        grid=(_GRID,),
        in_specs=[
            pl.BlockSpec((NC, _BR, D_H), lambda i: (0, i, 0)),
            pl.BlockSpec((NC, 2, _BR, HL), lambda i: (0, 0, i, 0)),
            pl.BlockSpec((1, D_H), lambda i: (0, 0)),
            pl.BlockSpec((D_H, D_OUT), lambda i: (0, 0)),
            pl.BlockSpec((1, D_OUT), lambda i: (0, 0)),
        ],
        out_specs=pl.BlockSpec((_BR, D_OUT), lambda i: (i, 0)),
        out_shape=jax.ShapeDtypeStruct((N, D_OUT), jnp.float32),
    )(agg, degs, b, wc, bc)


def kernel(x, edge_index, W1, b1, W2, b2, Wc, bc):
    src = edge_index[0].astype(jnp.int32)
    dst = edge_index[1].astype(jnp.int32)
    srcr = src.reshape(NC, NS, NCHUNK, K)
    dstr = dst.reshape(NC, NS, NCHUNK, K)

    hist_kernel, agg_kernel = _sc_kernels()
    degs = hist_kernel(srcr, dstr)
    hs1 = _mm1(x, degs, W1)
    agg1 = agg_kernel(hs1, srcr, dstr)
    hs2 = _mid(agg1, degs, b1.reshape(1, D_H), W2)
    agg2 = agg_kernel(hs2, srcr, dstr)
    return _fin(agg2, degs, b2.reshape(1, D_H), Wc, bc.reshape(1, D_OUT))
